# baseline (speedup 1.0000x reference)
import functools

import jax
import jax.numpy as jnp
from jax import lax
from jax.experimental import pallas as pl
from jax.experimental.pallas import tpu as pltpu
from jax.experimental.pallas import tpu_sc as plsc

N = 10000
E = 320000
D = 128
EPS = 1e-5

NC = 2
NS = 16
NW = NC * NS
EPT = E // NW
CHUNK = 80
NCHUNK = EPT // CHUNK
NP = 10240
RPT = NP // NS

_mesh = plsc.VectorSubcoreMesh(
    core_axis_name="c", subcore_axis_name="s", num_cores=NC, num_subcores=NS)


UNROLL = 4
NB = NCHUNK // UNROLL


@functools.partial(
    pl.kernel,
    out_type=jax.ShapeDtypeStruct((NC, NP, D), jnp.float32),
    mesh=_mesh,
    scratch_types=[
        pltpu.VMEM((UNROLL, CHUNK), jnp.int32),
        pltpu.VMEM((UNROLL, CHUNK), jnp.float32),
        pltpu.VMEM((UNROLL, CHUNK, D), jnp.float32),
        pltpu.VMEM_SHARED((NP, D), jnp.float32),
    ] + [pltpu.SemaphoreType.DMA] * (2 * UNROLL),
)
def _sc_degree(dst_hbm, ew_hbm, zero_hbm, out_hbm, dst_sl, ew_sl, rows_v,
               acc_sh, *sems):
    isems = sems[:UNROLL]
    ssems = sems[UNROLL:]
    cid = lax.axis_index("c")
    sid = lax.axis_index("s")
    wid = cid * NS + sid
    base_e = wid * EPT
    row0 = sid * RPT

    pltpu.sync_copy(zero_hbm.at[pl.ds(row0, RPT), :],
                    acc_sh.at[pl.ds(row0, RPT), :])
    plsc.subcore_barrier()

    def fill(i):
        def grp(g2, _):
            wv = ew_sl[i, pl.ds(g2 * 16, 16)]
            for k in range(16):
                w = jnp.full((16,), wv[k], jnp.float32)
                j = g2 * 16 + k
                for cc in range(D // 16):
                    rows_v[i, j, pl.ds(cc * 16, 16)] = w
            return 0

        lax.fori_loop(0, CHUNK // 16, grp, 0)

    def body(g, _):
        c0 = g * UNROLL
        ids = []
        for i in range(UNROLL):
            eb = base_e + (c0 + i) * CHUNK
            ids.append([
                pltpu.async_copy(dst_hbm.at[pl.ds(eb, CHUNK)],
                                 dst_sl.at[i], isems[i]),
                pltpu.async_copy(ew_hbm.at[pl.ds(eb, CHUNK)],
                                 ew_sl.at[i], isems[i]),
            ])
        sds = []
        for i in range(UNROLL):
            for d_ in ids[i]:
                d_.wait()
            fill(i)
            sds.append(pltpu.async_copy(
                rows_v.at[i], acc_sh.at[dst_sl.at[i]], ssems[i], add=True))
        for d_ in sds:
            d_.wait()
        return 0

    lax.fori_loop(0, NB, body, 0)
    for c in range(NB * UNROLL, NCHUNK):
        eb = base_e + c * CHUNK
        pltpu.sync_copy(dst_hbm.at[pl.ds(eb, CHUNK)], dst_sl.at[0])
        pltpu.sync_copy(ew_hbm.at[pl.ds(eb, CHUNK)], ew_sl.at[0])
        fill(0)
        pltpu.sync_copy(rows_v.at[0], acc_sh.at[dst_sl.at[0]], add=True)
    plsc.subcore_barrier()
    pltpu.sync_copy(acc_sh.at[pl.ds(row0, RPT), :],
                    out_hbm.at[cid, pl.ds(row0, RPT), :])


@functools.partial(
    pl.kernel,
    out_type=jax.ShapeDtypeStruct((NC, NP, D), jnp.float32),
    mesh=_mesh,
    scratch_types=[
        pltpu.VMEM((UNROLL, CHUNK), jnp.int32),
        pltpu.VMEM((UNROLL, CHUNK), jnp.int32),
        pltpu.VMEM((UNROLL, CHUNK), jnp.float32),
        pltpu.VMEM((UNROLL, CHUNK, D), jnp.float32),
        pltpu.VMEM_SHARED((NP, D), jnp.float32),
    ] + [pltpu.SemaphoreType.DMA] * (3 * UNROLL),
)
def _sc_spmm(h_hbm, src_hbm, dst_hbm, ew_hbm, zero_hbm, out_hbm,
             src_sl, dst_sl, ew_sl, rows_v, acc_sh, *sems):
    isems = sems[:UNROLL]
    gsems = sems[UNROLL:2 * UNROLL]
    ssems = sems[2 * UNROLL:]
    cid = lax.axis_index("c")
    sid = lax.axis_index("s")
    wid = cid * NS + sid
    base_e = wid * EPT
    row0 = sid * RPT

    pltpu.sync_copy(zero_hbm.at[pl.ds(row0, RPT), :],
                    acc_sh.at[pl.ds(row0, RPT), :])
    plsc.subcore_barrier()

    def scale(i):
        def grp(g2, _):
            wv = ew_sl[i, pl.ds(g2 * 16, 16)]
            for k in range(16):
                w = jnp.full((16,), wv[k], jnp.float32)
                j = g2 * 16 + k
                for cc in range(D // 16):
                    sl = pl.ds(cc * 16, 16)
                    rows_v[i, j, sl] = rows_v[i, j, sl] * w
            return 0

        lax.fori_loop(0, CHUNK // 16, grp, 0)

    def body(g, _):
        c0 = g * UNROLL
        ids = []
        for i in range(UNROLL):
            eb = base_e + (c0 + i) * CHUNK
            ids.append([
                pltpu.async_copy(src_hbm.at[pl.ds(eb, CHUNK)],
                                 src_sl.at[i], isems[i]),
                pltpu.async_copy(dst_hbm.at[pl.ds(eb, CHUNK)],
                                 dst_sl.at[i], isems[i]),
                pltpu.async_copy(ew_hbm.at[pl.ds(eb, CHUNK)],
                                 ew_sl.at[i], isems[i]),
            ])
        gds = []
        for i in range(UNROLL):
            for d_ in ids[i]:
                d_.wait()
            gds.append(pltpu.async_copy(h_hbm.at[src_sl.at[i]],
                                        rows_v.at[i], gsems[i]))
        sds = []
        for i in range(UNROLL):
            gds[i].wait()
            scale(i)
            sds.append(pltpu.async_copy(
                rows_v.at[i], acc_sh.at[dst_sl.at[i]], ssems[i], add=True))
        for d_ in sds:
            d_.wait()
        return 0

    lax.fori_loop(0, NB, body, 0)
    for c in range(NB * UNROLL, NCHUNK):
        eb = base_e + c * CHUNK
        pltpu.sync_copy(src_hbm.at[pl.ds(eb, CHUNK)], src_sl.at[0])
        pltpu.sync_copy(dst_hbm.at[pl.ds(eb, CHUNK)], dst_sl.at[0])
        pltpu.sync_copy(ew_hbm.at[pl.ds(eb, CHUNK)], ew_sl.at[0])
        pltpu.async_copy(h_hbm.at[src_sl.at[0]], rows_v.at[0], gsems[0]).wait()
        scale(0)
        pltpu.sync_copy(rows_v.at[0], acc_sh.at[dst_sl.at[0]], add=True)
    plsc.subcore_barrier()
    pltpu.sync_copy(acc_sh.at[pl.ds(row0, RPT), :],
                    out_hbm.at[cid, pl.ds(row0, RPT), :])


BN_ROWS = 1000
GRID = N // BN_ROWS


def _row_spec(width=D):
    return pl.BlockSpec((BN_ROWS, width), lambda i: (i, 0))


def _full_spec(shape):
    return pl.BlockSpec(shape, lambda i: tuple(0 for _ in shape))


def _tc_mm_plain_body(x_ref, w_ref, g_ref):
    g_ref[...] = jnp.dot(x_ref[...], w_ref[...],
                         preferred_element_type=jnp.float32)


def _tc_mm_plain(x, w):
    return pl.pallas_call(
        _tc_mm_plain_body,
        grid=(GRID,),
        in_specs=[_row_spec(), _full_spec((D, D))],
        out_specs=_row_spec(),
        out_shape=jax.ShapeDtypeStruct((N, D), jnp.float32),
    )(x, w)


def _tc_scale0_body(g_ref, dga_ref, dgb_ref, h_ref, dinv_ref):
    deg = dga_ref[...] + dgb_ref[...] + 1.0
    dinv = jnp.where(deg > 0, lax.rsqrt(jnp.maximum(deg, 1e-12)), 0.0)
    dinv_ref[...] = dinv
    h_ref[...] = dinv[:, 0:1] * g_ref[...]


def _tc_scale0(g, dga, dgb):
    return pl.pallas_call(
        _tc_scale0_body,
        grid=(GRID,),
        in_specs=[_row_spec(), _row_spec(), _row_spec()],
        out_specs=[_row_spec(), _row_spec()],
        out_shape=[jax.ShapeDtypeStruct((N, D), jnp.float32),
                   jax.ShapeDtypeStruct((N, D), jnp.float32)],
    )(g, dga, dgb)


def _tc_post_body(pa_ref, pb_ref, h_ref, x0_ref, dinv_ref, b_ref,
                  y_ref, st_ref, acc_ref):
    i = pl.program_id(0)

    @pl.when(i == 0)
    def _():
        acc_ref[...] = jnp.zeros_like(acc_ref)

    dinv = dinv_ref[:, 0:1]
    agg = dinv * (pa_ref[...] + pb_ref[...] + h_ref[...]) + b_ref[...]
    y = jax.nn.relu(agg) + x0_ref[...]
    y_ref[...] = y
    acc_ref[0:1, :] += jnp.sum(y, axis=0, keepdims=True)
    acc_ref[1:2, :] += jnp.sum(y * y, axis=0, keepdims=True)

    @pl.when(i == GRID - 1)
    def _():
        st_ref[...] = acc_ref[...]


def _tc_post(pa, pb, h, x0, dinv, b):
    return pl.pallas_call(
        _tc_post_body,
        grid=(GRID,),
        in_specs=[_row_spec(), _row_spec(), _row_spec(), _row_spec(),
                  _row_spec(), _full_spec((1, D))],
        out_specs=[_row_spec(), _full_spec((8, D))],
        out_shape=[jax.ShapeDtypeStruct((N, D), jnp.float32),
                   jax.ShapeDtypeStruct((8, D), jnp.float32)],
        scratch_shapes=[pltpu.VMEM((8, D), jnp.float32)],
    )(pa, pb, h, x0, dinv, b)


def _bn_apply(y, st_ref, gamma_ref, beta_ref):
    mean = st_ref[0:1, :] * (1.0 / N)
    var = st_ref[1:2, :] * (1.0 / N) - mean * mean
    return (y - mean) * lax.rsqrt(var + EPS) * gamma_ref[...] + beta_ref[...]


def _tc_bn_mm_body(y_ref, st_ref, g_ref, bt_ref, w_ref, dinv_ref, h_ref):
    xn = _bn_apply(y_ref[...], st_ref, g_ref, bt_ref)
    h_ref[...] = dinv_ref[:, 0:1] * jnp.dot(
        xn, w_ref[...], preferred_element_type=jnp.float32)


def _tc_bn_mm(y, st, g, bt, w, dinv):
    return pl.pallas_call(
        _tc_bn_mm_body,
        grid=(GRID,),
        in_specs=[_row_spec(), _full_spec((8, D)), _full_spec((1, D)),
                  _full_spec((1, D)), _full_spec((D, D)), _row_spec()],
        out_specs=_row_spec(),
        out_shape=jax.ShapeDtypeStruct((N, D), jnp.float32),
    )(y, st, g, bt, w, dinv)


def _tc_bn_body(y_ref, st_ref, g_ref, bt_ref, o_ref):
    o_ref[...] = _bn_apply(y_ref[...], st_ref, g_ref, bt_ref)


def _tc_bn(y, st, g, bt):
    return pl.pallas_call(
        _tc_bn_body,
        grid=(GRID,),
        in_specs=[_row_spec(), _full_spec((8, D)), _full_spec((1, D)),
                  _full_spec((1, D))],
        out_specs=_row_spec(),
        out_shape=jax.ShapeDtypeStruct((N, D), jnp.float32),
    )(y, st, g, bt)


def kernel(decoded_reshaped_x, edge_index, edge_weight,
           W0, b0, gamma0, beta0,
           W1, b1, gamma1, beta1,
           W2, b2, gamma2, beta2):
    x0 = decoded_reshaped_x
    src = edge_index[0]
    dst = edge_index[1]
    ew3 = edge_weight
    zeros_big = jnp.zeros((NP, D), jnp.float32)

    degp = _sc_degree(dst, ew3, zeros_big)
    g0 = _tc_mm_plain(x0, W0)
    dga = degp[0, :N]
    dgb = degp[1, :N]

    Ws = (W0, W1, W2)
    bs = (b0.reshape(1, D), b1.reshape(1, D), b2.reshape(1, D))
    gs = (gamma0.reshape(1, D), gamma1.reshape(1, D), gamma2.reshape(1, D))
    bts = (beta0.reshape(1, D), beta1.reshape(1, D), beta2.reshape(1, D))

    h, dinv = _tc_scale0(g0, dga, dgb)
    y = None
    st = None
    for i in range(3):
        p = _sc_spmm(h, src, dst, ew3, zeros_big)
        y, st = _tc_post(p[0, :N], p[1, :N], h, x0, dinv, bs[i])
        if i < 2:
            h = _tc_bn_mm(y, st, gs[i], bts[i], Ws[i + 1], dinv)
    return _tc_bn(y, st, gs[2], bts[2])


# speedup vs baseline: 1.0104x; 1.0104x over previous
import functools

import jax
import jax.numpy as jnp
from jax import lax
from jax.experimental import pallas as pl
from jax.experimental.pallas import tpu as pltpu
from jax.experimental.pallas import tpu_sc as plsc

N = 10000
E = 320000
D = 128
EPS = 1e-5

NC = 2
NS = 16
NW = NC * NS
EPT = E // NW
CHUNK = 80
NCHUNK = EPT // CHUNK
NP = 10240
RPT = NP // NS

_mesh = plsc.VectorSubcoreMesh(
    core_axis_name="c", subcore_axis_name="s", num_cores=NC, num_subcores=NS)


UNROLL = 4
NB = NCHUNK // UNROLL


@functools.partial(
    pl.kernel,
    out_type=jax.ShapeDtypeStruct((NC, NP, D), jnp.float32),
    mesh=_mesh,
    scratch_types=[
        pltpu.VMEM((UNROLL, CHUNK), jnp.int32),
        pltpu.VMEM((UNROLL, CHUNK), jnp.float32),
        pltpu.VMEM((UNROLL, CHUNK, D), jnp.float32),
        pltpu.VMEM_SHARED((NP, D), jnp.float32),
    ] + [pltpu.SemaphoreType.DMA] * (2 * UNROLL),
)
def _sc_degree(dst_hbm, ew_hbm, zero_hbm, out_hbm, dst_sl, ew_sl, rows_v,
               acc_sh, *sems):
    isems = sems[:UNROLL]
    ssems = sems[UNROLL:]
    cid = lax.axis_index("c")
    sid = lax.axis_index("s")
    wid = cid * NS + sid
    base_e = wid * EPT
    row0 = sid * RPT

    pltpu.sync_copy(zero_hbm.at[pl.ds(row0, RPT), :],
                    acc_sh.at[pl.ds(row0, RPT), :])
    plsc.subcore_barrier()

    def fill(i):
        def grp(g2, _):
            wv = ew_sl[i, pl.ds(g2 * 16, 16)]
            for k in range(16):
                w = jnp.full((16,), wv[k], jnp.float32)
                j = g2 * 16 + k
                for cc in range(D // 16):
                    rows_v[i, j, pl.ds(cc * 16, 16)] = w
            return 0

        lax.fori_loop(0, CHUNK // 16, grp, 0)

    def body(g, _):
        c0 = g * UNROLL
        ids = []
        for i in range(UNROLL):
            eb = base_e + (c0 + i) * CHUNK
            ids.append([
                pltpu.async_copy(dst_hbm.at[pl.ds(eb, CHUNK)],
                                 dst_sl.at[i], isems[i]),
                pltpu.async_copy(ew_hbm.at[pl.ds(eb, CHUNK)],
                                 ew_sl.at[i], isems[i]),
            ])
        sds = []
        for i in range(UNROLL):
            for d_ in ids[i]:
                d_.wait()
            fill(i)
            sds.append(pltpu.async_copy(
                rows_v.at[i], acc_sh.at[dst_sl.at[i]], ssems[i], add=True))
        for d_ in sds:
            d_.wait()
        return 0

    lax.fori_loop(0, NB, body, 0)
    for c in range(NB * UNROLL, NCHUNK):
        eb = base_e + c * CHUNK
        pltpu.sync_copy(dst_hbm.at[pl.ds(eb, CHUNK)], dst_sl.at[0])
        pltpu.sync_copy(ew_hbm.at[pl.ds(eb, CHUNK)], ew_sl.at[0])
        fill(0)
        pltpu.sync_copy(rows_v.at[0], acc_sh.at[dst_sl.at[0]], add=True)
    plsc.subcore_barrier()
    pltpu.sync_copy(acc_sh.at[pl.ds(row0, RPT), :],
                    out_hbm.at[cid, pl.ds(row0, RPT), :])


@functools.partial(
    pl.kernel,
    out_type=jax.ShapeDtypeStruct((NC, NP, D), jnp.float32),
    mesh=_mesh,
    scratch_types=[
        pltpu.VMEM((UNROLL, CHUNK), jnp.int32),
        pltpu.VMEM((UNROLL, CHUNK), jnp.int32),
        pltpu.VMEM((UNROLL, CHUNK), jnp.float32),
        pltpu.VMEM((UNROLL, CHUNK, D), jnp.float32),
        pltpu.VMEM_SHARED((NP, D), jnp.float32),
    ] + [pltpu.SemaphoreType.DMA] * (3 * UNROLL),
)
def _sc_spmm(h_hbm, src_hbm, dst_hbm, ew_hbm, zero_hbm, out_hbm,
             src_sl, dst_sl, ew_sl, rows_v, acc_sh, *sems):
    isems = sems[:UNROLL]
    gsems = sems[UNROLL:2 * UNROLL]
    ssems = sems[2 * UNROLL:]
    cid = lax.axis_index("c")
    sid = lax.axis_index("s")
    wid = cid * NS + sid
    base_e = wid * EPT
    row0 = sid * RPT

    pltpu.sync_copy(zero_hbm.at[pl.ds(row0, RPT), :],
                    acc_sh.at[pl.ds(row0, RPT), :])
    plsc.subcore_barrier()

    def scale(i):
        def grp(g2, _):
            wv = ew_sl[i, pl.ds(g2 * 16, 16)]
            for k in range(16):
                w = jnp.full((16,), wv[k], jnp.float32)
                j = g2 * 16 + k
                for cc in range(D // 16):
                    sl = pl.ds(cc * 16, 16)
                    rows_v[i, j, sl] = rows_v[i, j, sl] * w
            return 0

        lax.fori_loop(0, CHUNK // 16, grp, 0)

    def body(g, _):
        c0 = g * UNROLL
        ids = []
        for i in range(UNROLL):
            eb = base_e + (c0 + i) * CHUNK
            ids.append([
                pltpu.async_copy(src_hbm.at[pl.ds(eb, CHUNK)],
                                 src_sl.at[i], isems[i]),
                pltpu.async_copy(dst_hbm.at[pl.ds(eb, CHUNK)],
                                 dst_sl.at[i], isems[i]),
                pltpu.async_copy(ew_hbm.at[pl.ds(eb, CHUNK)],
                                 ew_sl.at[i], isems[i]),
            ])
        gds = []
        for i in range(UNROLL):
            for d_ in ids[i]:
                d_.wait()
            gds.append(pltpu.async_copy(h_hbm.at[src_sl.at[i]],
                                        rows_v.at[i], gsems[i]))
        sds = []
        for i in range(UNROLL):
            gds[i].wait()
            scale(i)
            sds.append(pltpu.async_copy(
                rows_v.at[i], acc_sh.at[dst_sl.at[i]], ssems[i], add=True))
        for d_ in sds:
            d_.wait()
        return 0

    lax.fori_loop(0, NB, body, 0)
    for c in range(NB * UNROLL, NCHUNK):
        eb = base_e + c * CHUNK
        pltpu.sync_copy(src_hbm.at[pl.ds(eb, CHUNK)], src_sl.at[0])
        pltpu.sync_copy(dst_hbm.at[pl.ds(eb, CHUNK)], dst_sl.at[0])
        pltpu.sync_copy(ew_hbm.at[pl.ds(eb, CHUNK)], ew_sl.at[0])
        pltpu.async_copy(h_hbm.at[src_sl.at[0]], rows_v.at[0], gsems[0]).wait()
        scale(0)
        pltpu.sync_copy(rows_v.at[0], acc_sh.at[dst_sl.at[0]], add=True)
    plsc.subcore_barrier()
    pltpu.sync_copy(acc_sh.at[pl.ds(row0, RPT), :],
                    out_hbm.at[cid, pl.ds(row0, RPT), :])


BN_ROWS = 1000
GRID = N // BN_ROWS


def _row_spec(width=D):
    return pl.BlockSpec((BN_ROWS, width), lambda i: (i, 0))


def _full_spec(shape):
    return pl.BlockSpec(shape, lambda i: tuple(0 for _ in shape))


def _tc_mm_plain_body(x_ref, w_ref, g_ref):
    g_ref[...] = jnp.dot(x_ref[...], w_ref[...],
                         preferred_element_type=jnp.float32)


def _tc_mm_plain(x, w):
    return pl.pallas_call(
        _tc_mm_plain_body,
        grid=(GRID,),
        in_specs=[_row_spec(), _full_spec((D, D))],
        out_specs=_row_spec(),
        out_shape=jax.ShapeDtypeStruct((N, D), jnp.float32),
    )(x, w)


def _tc_scale0_body(g_ref, dga_ref, dgb_ref, h_ref, dinv_ref):
    deg = dga_ref[...] + dgb_ref[...] + 1.0
    dinv = jnp.where(deg > 0, lax.rsqrt(jnp.maximum(deg, 1e-12)), 0.0)
    dinv_ref[...] = dinv
    h_ref[...] = dinv[:, 0:1] * g_ref[...]


def _tc_scale0(g, dga, dgb):
    return pl.pallas_call(
        _tc_scale0_body,
        grid=(GRID,),
        in_specs=[_row_spec(), _row_spec(16), _row_spec(16)],
        out_specs=[_row_spec(), _row_spec(16)],
        out_shape=[jax.ShapeDtypeStruct((N, D), jnp.float32),
                   jax.ShapeDtypeStruct((N, 16), jnp.float32)],
    )(g, dga, dgb)


def _tc_layer_body(pa_ref, pb_ref, h_ref, x0_ref, dinv_ref, b_ref,
                   g_ref, bt_ref, w_ref, hn_ref, y_sc, acc_ref, *, final):
    p = pl.program_id(0)
    i = pl.program_id(1)

    @pl.when(p == 0)
    def _():
        @pl.when(i == 0)
        def _():
            acc_ref[...] = jnp.zeros_like(acc_ref)

        dinv = dinv_ref[:, 0:1]
        agg = dinv * (pa_ref[...] + pb_ref[...] + h_ref[...]) + b_ref[...]
        y = jax.nn.relu(agg) + x0_ref[...]
        r0 = pl.multiple_of(i * BN_ROWS, BN_ROWS)
        y_sc[pl.ds(r0, BN_ROWS), :] = y
        acc_ref[0:1, :] += jnp.sum(y, axis=0, keepdims=True)
        acc_ref[1:2, :] += jnp.sum(y * y, axis=0, keepdims=True)

    @pl.when(p == 1)
    def _():
        mean = acc_ref[0:1, :] * (1.0 / N)
        var = acc_ref[1:2, :] * (1.0 / N) - mean * mean
        r0 = pl.multiple_of(i * BN_ROWS, BN_ROWS)
        y = y_sc[pl.ds(r0, BN_ROWS), :]
        xn = (y - mean) * lax.rsqrt(var + EPS) * g_ref[...] + bt_ref[...]
        if final:
            hn_ref[...] = xn
        else:
            hn_ref[...] = dinv_ref[:, 0:1] * jnp.dot(
                xn, w_ref[...], preferred_element_type=jnp.float32)


def _tc_layer(pa, pb, h, x0, dinv, b, g, bt, w, final):
    ph0_row = pl.BlockSpec((BN_ROWS, D), lambda p, i: (i * (1 - p), 0))
    body = functools.partial(_tc_layer_body, final=final)
    return pl.pallas_call(
        body,
        grid=(2, GRID),
        in_specs=[ph0_row, ph0_row, ph0_row, ph0_row,
                  pl.BlockSpec((BN_ROWS, 16), lambda p, i: (i, 0)),
                  pl.BlockSpec((1, D), lambda p, i: (0, 0)),
                  pl.BlockSpec((1, D), lambda p, i: (0, 0)),
                  pl.BlockSpec((1, D), lambda p, i: (0, 0)),
                  pl.BlockSpec((D, D), lambda p, i: (0, 0))],
        out_specs=pl.BlockSpec((BN_ROWS, D), lambda p, i: (i * p, 0)),
        out_shape=jax.ShapeDtypeStruct((N, D), jnp.float32),
        scratch_shapes=[pltpu.VMEM((N, D), jnp.float32),
                        pltpu.VMEM((8, D), jnp.float32)],
    )(pa, pb, h, x0, dinv, b, g, bt, w)


def kernel(decoded_reshaped_x, edge_index, edge_weight,
           W0, b0, gamma0, beta0,
           W1, b1, gamma1, beta1,
           W2, b2, gamma2, beta2):
    x0 = decoded_reshaped_x
    src = edge_index[0]
    dst = edge_index[1]
    ew3 = edge_weight
    zeros_big = jnp.zeros((NP, D), jnp.float32)

    degp = _sc_degree(dst, ew3, zeros_big)
    g0 = _tc_mm_plain(x0, W0)
    dga = degp[0, :N, :16]
    dgb = degp[1, :N, :16]

    Ws = (W0, W1, W2)
    bs = (b0.reshape(1, D), b1.reshape(1, D), b2.reshape(1, D))
    gs = (gamma0.reshape(1, D), gamma1.reshape(1, D), gamma2.reshape(1, D))
    bts = (beta0.reshape(1, D), beta1.reshape(1, D), beta2.reshape(1, D))

    h, dinv = _tc_scale0(g0, dga, dgb)
    out = None
    for i in range(3):
        p = _sc_spmm(h, src, dst, ew3, zeros_big)
        nxt = _tc_layer(p[0, :N], p[1, :N], h, x0, dinv, bs[i], gs[i],
                        bts[i], Ws[min(i + 1, 2)], final=(i == 2))
        if i < 2:
            h = nxt
        else:
            out = nxt
    return out
